# Initial kernel scaffold; baseline (speedup 1.0000x reference)
import functools

import jax
import jax.numpy as jnp
from jax import lax
from jax.experimental import pallas as pl
from jax.experimental.pallas import tpu as pltpu
from jax.experimental.pallas import tpu_sc as plsc

N = 10000
E = 320000
D = 128

NC = 2
NS = 16
NW = NC * NS

EROWS = E // 128
ROWS_PER_W = EROWS // NW
EXTRA_ROWS = EROWS - ROWS_PER_W * NW

H_CHUNK = 624
H_LAST = N - 15 * H_CHUNK

RPT = N // NS

_mesh = plsc.VectorSubcoreMesh(core_axis_name="c", subcore_axis_name="s")


@functools.partial(
    pl.kernel,
    out_type=jax.ShapeDtypeStruct((NC, 2, N), jnp.float32),
    mesh=_mesh,
    scratch_types=[
        pltpu.VMEM((13, 128), jnp.int32),
        pltpu.VMEM((13, 128), jnp.int32),
        pltpu.VMEM((128,), jnp.float32),
        pltpu.VMEM_SHARED((N,), jnp.float32),
        pltpu.VMEM_SHARED((N,), jnp.float32),
        pltpu.SemaphoreType.DMA,
    ],
)
def _hist_sc(s2d, r2d, ones_hbm, z1d_hbm, out, sidx_v, ridx_v, ones_v,
             hs_sh, hr_sh, sem):
    c = lax.axis_index("c")
    s = lax.axis_index("s")
    g = s * NC + c

    off = s * H_CHUNK

    @pl.when(s == NS - 1)
    def _():
        pltpu.sync_copy(z1d_hbm, hs_sh.at[pl.ds(off, H_LAST)])
        pltpu.sync_copy(z1d_hbm, hr_sh.at[pl.ds(off, H_LAST)])

    @pl.when(s != NS - 1)
    def _():
        pltpu.sync_copy(z1d_hbm.at[pl.ds(0, H_CHUNK)],
                        hs_sh.at[pl.ds(off, H_CHUNK)])
        pltpu.sync_copy(z1d_hbm.at[pl.ds(0, H_CHUNK)],
                        hr_sh.at[pl.ds(off, H_CHUNK)])

    pltpu.sync_copy(ones_hbm, ones_v)
    plsc.subcore_barrier()

    row0 = g * ROWS_PER_W

    def body(i, _):
        base = row0 + i * 13
        pltpu.sync_copy(s2d.at[pl.ds(base, 13)], sidx_v)
        pltpu.sync_copy(r2d.at[pl.ds(base, 13)], ridx_v)
        for j in range(13):
            pltpu.sync_copy(ones_v, hs_sh.at[sidx_v.at[j]], add=True)
            pltpu.sync_copy(ones_v, hr_sh.at[ridx_v.at[j]], add=True)
        return 0

    lax.fori_loop(0, ROWS_PER_W // 13, body, 0)

    @pl.when(g == 0)
    def _():
        base = EROWS - EXTRA_ROWS
        pltpu.sync_copy(s2d.at[pl.ds(base, EXTRA_ROWS)],
                        sidx_v.at[pl.ds(0, EXTRA_ROWS)])
        pltpu.sync_copy(r2d.at[pl.ds(base, EXTRA_ROWS)],
                        ridx_v.at[pl.ds(0, EXTRA_ROWS)])
        for j in range(EXTRA_ROWS):
            pltpu.sync_copy(ones_v, hs_sh.at[sidx_v.at[j]], add=True)
            pltpu.sync_copy(ones_v, hr_sh.at[ridx_v.at[j]], add=True)

    plsc.subcore_barrier()

    @pl.when(s == NS - 1)
    def _():
        pltpu.sync_copy(hs_sh.at[pl.ds(off, H_LAST)],
                        out.at[c, 0, pl.ds(off, H_LAST)])
        pltpu.sync_copy(hr_sh.at[pl.ds(off, H_LAST)],
                        out.at[c, 1, pl.ds(off, H_LAST)])

    @pl.when(s != NS - 1)
    def _():
        pltpu.sync_copy(hs_sh.at[pl.ds(off, H_CHUNK)],
                        out.at[c, 0, pl.ds(off, H_CHUNK)])
        pltpu.sync_copy(hr_sh.at[pl.ds(off, H_CHUNK)],
                        out.at[c, 1, pl.ds(off, H_CHUNK)])


def _matmul_scale_body(nodes_ref, w_ref, hs0_ref, hs1_ref, hr0_ref, hr1_ref,
                       out_ref):
    cs = hs0_ref[...] + hs1_ref[...]
    cr = hr0_ref[...] + hr1_ref[...]
    scale = lax.rsqrt((2.0 * cs + 2.0) * (2.0 * cr + 2.0))
    x = jnp.dot(nodes_ref[...], w_ref[...], preferred_element_type=jnp.float32)
    out_ref[...] = x * scale


def _matmul_scale_tc(nodes, W, hs0, hs1, hr0, hr1):
    BR = 500
    grid = (N // BR,)
    return pl.pallas_call(
        _matmul_scale_body,
        grid=grid,
        in_specs=[
            pl.BlockSpec((BR, D), lambda i: (i, 0)),
            pl.BlockSpec((D, D), lambda i: (0, 0)),
            pl.BlockSpec((BR, 1), lambda i: (i, 0)),
            pl.BlockSpec((BR, 1), lambda i: (i, 0)),
            pl.BlockSpec((BR, 1), lambda i: (i, 0)),
            pl.BlockSpec((BR, 1), lambda i: (i, 0)),
        ],
        out_specs=pl.BlockSpec((BR, D), lambda i: (i, 0)),
        out_shape=jax.ShapeDtypeStruct((N, D), jnp.float32),
    )(nodes, W, hs0, hs1, hr0, hr1)


CH = 6


@functools.partial(
    pl.kernel,
    out_type=(jax.ShapeDtypeStruct((N, D), jnp.float32),
              jax.ShapeDtypeStruct((N, D), jnp.float32)),
    mesh=_mesh,
    scratch_types=[
        pltpu.VMEM((13, 128), jnp.int32),
        pltpu.VMEM((13, 128), jnp.int32),
        pltpu.VMEM((CH, 128, D), jnp.float32),
        pltpu.VMEM_SHARED((N, D), jnp.float32),
        pltpu.SemaphoreType.DMA,
    ],
)
def _message_sc(xs_hbm, s2d, r2d, z2d_hbm, out_a, out_b,
                sidx_v, ridx_v, rows_v, acc_sh, sem):
    c = lax.axis_index("c")
    s = lax.axis_index("s")
    g = s * NC + c

    row0 = s * RPT

    @pl.when(c == 0)
    def _():
        pltpu.sync_copy(xs_hbm.at[pl.ds(row0, RPT)],
                        acc_sh.at[pl.ds(row0, RPT)])

    @pl.when(c == 1)
    def _():
        for k in range(9):
            pltpu.sync_copy(z2d_hbm, acc_sh.at[pl.ds(row0 + k * 64, 64)])
        pltpu.sync_copy(z2d_hbm.at[pl.ds(0, RPT - 9 * 64)],
                        acc_sh.at[pl.ds(row0 + 9 * 64, RPT - 9 * 64)])

    plsc.subcore_barrier()

    erow0 = g * ROWS_PER_W

    def body(i, _):
        base = erow0 + i * CH
        pltpu.sync_copy(s2d.at[pl.ds(base, CH)], sidx_v.at[pl.ds(0, CH)])
        pltpu.sync_copy(r2d.at[pl.ds(base, CH)], ridx_v.at[pl.ds(0, CH)])
        cps = [pltpu.async_copy(xs_hbm.at[sidx_v.at[j]], rows_v.at[j], sem)
               for j in range(CH)]
        for cp in cps:
            cp.wait()
        for j in range(CH):
            pltpu.sync_copy(rows_v.at[j], acc_sh.at[ridx_v.at[j]], add=True)
        return 0

    lax.fori_loop(0, ROWS_PER_W // CH, body, 0)

    @pl.when(g == 0)
    def _():
        base = EROWS - EXTRA_ROWS
        pltpu.sync_copy(s2d.at[pl.ds(base, EXTRA_ROWS)],
                        sidx_v.at[pl.ds(0, EXTRA_ROWS)])
        pltpu.sync_copy(r2d.at[pl.ds(base, EXTRA_ROWS)],
                        ridx_v.at[pl.ds(0, EXTRA_ROWS)])
        cps = [pltpu.async_copy(xs_hbm.at[sidx_v.at[j]], rows_v.at[j], sem)
               for j in range(EXTRA_ROWS)]
        for cp in cps:
            cp.wait()
        for j in range(EXTRA_ROWS):
            pltpu.sync_copy(rows_v.at[j], acc_sh.at[ridx_v.at[j]], add=True)

    plsc.subcore_barrier()

    @pl.when(c == 0)
    def _():
        pltpu.sync_copy(acc_sh.at[pl.ds(row0, RPT)],
                        out_a.at[pl.ds(row0, RPT)])

    @pl.when(c == 1)
    def _():
        pltpu.sync_copy(acc_sh.at[pl.ds(row0, RPT)],
                        out_b.at[pl.ds(row0, RPT)])


def _add_body(a_ref, b_ref, out_ref):
    out_ref[...] = a_ref[...] + b_ref[...]


def _add_tc(a, b):
    BR = 1000
    return pl.pallas_call(
        _add_body,
        grid=(N // BR,),
        in_specs=[pl.BlockSpec((BR, D), lambda i: (i, 0)),
                  pl.BlockSpec((BR, D), lambda i: (i, 0))],
        out_specs=pl.BlockSpec((BR, D), lambda i: (i, 0)),
        out_shape=jax.ShapeDtypeStruct((N, D), jnp.float32),
    )(a, b)


def kernel(nodes, senders, receivers, W):
    s2d = senders.reshape(EROWS, 128)
    r2d = receivers.reshape(EROWS, 128)
    ones128 = jnp.ones((128,), jnp.float32)
    z1d = jnp.zeros((H_LAST,), jnp.float32)
    z2d = jnp.zeros((64, D), jnp.float32)

    hist = _hist_sc(s2d, r2d, ones128, z1d)
    hs0 = hist[0, 0].reshape(N, 1)
    hs1 = hist[1, 0].reshape(N, 1)
    hr0 = hist[0, 1].reshape(N, 1)
    hr1 = hist[1, 1].reshape(N, 1)

    xs = _matmul_scale_tc(nodes, W, hs0, hs1, hr0, hr1)
    pa, pb = _message_sc(xs, s2d, r2d, z2d)
    return _add_tc(pa, pb)


# speedup vs baseline: 4.8406x; 4.8406x over previous
import functools

import jax
import jax.numpy as jnp
from jax import lax
from jax.experimental import pallas as pl
from jax.experimental.pallas import tpu as pltpu
from jax.experimental.pallas import tpu_sc as plsc

N = 10000
E = 320000
D = 128

NC = 2
NS = 16
NW = NC * NS

EROWS = 2560
EPAD = EROWS * 128 - E
RW = EROWS // NW
CHUNK = 8
RB = 2

NB = N + 16
HB = 10240
HT = HB // NS

T_CHUNK = 624
T_LAST = N - 15 * T_CHUNK

_mesh = plsc.VectorSubcoreMesh(core_axis_name="c", subcore_axis_name="s",
                               num_cores=NC, num_subcores=NS)


@functools.partial(
    pl.kernel,
    out_type=(jax.ShapeDtypeStruct((HB,), jnp.float32),
              jax.ShapeDtypeStruct((HB,), jnp.float32),
              jax.ShapeDtypeStruct((HB,), jnp.float32),
              jax.ShapeDtypeStruct((HB,), jnp.float32)),
    mesh=_mesh,
    scratch_types=[
        pltpu.VMEM((CHUNK, 128), jnp.int32),
        pltpu.VMEM((CHUNK, 128), jnp.int32),
        pltpu.VMEM((128,), jnp.float32),
        pltpu.VMEM_SHARED((HB,), jnp.float32),
        pltpu.VMEM_SHARED((HB,), jnp.float32),
        pltpu.SemaphoreType.DMA,
    ],
)
def _hist_sc(s2d, r2d, ones_hbm, z1d_hbm, hs_a, hr_a, hs_b, hr_b,
             sidx_v, ridx_v, ones_v, hs_sh, hr_sh, sem):
    c = lax.axis_index("c")
    s = lax.axis_index("s")
    g = s * NC + c

    off = s * HT
    pltpu.sync_copy(z1d_hbm, hs_sh.at[pl.ds(off, HT)])
    pltpu.sync_copy(z1d_hbm, hr_sh.at[pl.ds(off, HT)])
    pltpu.sync_copy(ones_hbm, ones_v)
    plsc.subcore_barrier()

    row0 = g * RW

    def body(i, _):
        base = row0 + i * CHUNK
        pltpu.sync_copy(s2d.at[pl.ds(base, CHUNK)], sidx_v)
        pltpu.sync_copy(r2d.at[pl.ds(base, CHUNK)], ridx_v)
        for j in range(CHUNK):
            pltpu.sync_copy(ones_v, hs_sh.at[sidx_v.at[j]], add=True)
            pltpu.sync_copy(ones_v, hr_sh.at[ridx_v.at[j]], add=True)
        return 0

    lax.fori_loop(0, RW // CHUNK, body, 0)
    plsc.subcore_barrier()

    @pl.when(c == 0)
    def _():
        pltpu.sync_copy(hs_sh.at[pl.ds(off, HT)], hs_a.at[pl.ds(off, HT)])
        pltpu.sync_copy(hr_sh.at[pl.ds(off, HT)], hr_a.at[pl.ds(off, HT)])

    @pl.when(c == 1)
    def _():
        pltpu.sync_copy(hs_sh.at[pl.ds(off, HT)], hs_b.at[pl.ds(off, HT)])
        pltpu.sync_copy(hr_sh.at[pl.ds(off, HT)], hr_b.at[pl.ds(off, HT)])


def _matmul_scale_body(nodes_ref, w_ref, hs0_ref, hs1_ref, hr0_ref, hr1_ref,
                       out_ref):
    cs = hs0_ref[...] + hs1_ref[...]
    cr = hr0_ref[...] + hr1_ref[...]
    scale = lax.rsqrt((2.0 * cs + 2.0) * (2.0 * cr + 2.0))
    x = jnp.dot(nodes_ref[...], w_ref[...], preferred_element_type=jnp.float32)
    out_ref[...] = x * scale


def _matmul_scale_tc(nodes, W, hs0, hs1, hr0, hr1):
    BR = 1000
    grid = (N // BR,)
    return pl.pallas_call(
        _matmul_scale_body,
        grid=grid,
        in_specs=[
            pl.BlockSpec((BR, D), lambda i: (i, 0)),
            pl.BlockSpec((D, D), lambda i: (0, 0)),
            pl.BlockSpec((BR, 1), lambda i: (i, 0)),
            pl.BlockSpec((BR, 1), lambda i: (i, 0)),
            pl.BlockSpec((BR, 1), lambda i: (i, 0)),
            pl.BlockSpec((BR, 1), lambda i: (i, 0)),
        ],
        out_specs=pl.BlockSpec((BR, D), lambda i: (i, 0)),
        out_shape=jax.ShapeDtypeStruct((N, D), jnp.float32),
    )(nodes, W, hs0, hs1, hr0, hr1)


@functools.partial(
    pl.kernel,
    out_type=(jax.ShapeDtypeStruct((N, D), jnp.float32),
              jax.ShapeDtypeStruct((N, D), jnp.float32)),
    mesh=_mesh,
    scratch_types=[
        pltpu.VMEM((CHUNK, 128), jnp.int32),
        pltpu.VMEM((CHUNK, 128), jnp.int32),
        pltpu.VMEM((RB, 128, D), jnp.float32),
        pltpu.VMEM_SHARED((NB, D), jnp.float32),
        pltpu.SemaphoreType.DMA,
        pltpu.SemaphoreType.DMA,
    ],
)
def _message_sc(xs_hbm, s2d, r2d, z2d_hbm, out_a, out_b,
                sidx_v, ridx_v, rows_v, acc_sh, gsem, ssem):
    c = lax.axis_index("c")
    s = lax.axis_index("s")
    g = s * NC + c

    off = s * T_CHUNK

    @pl.when((c == 0) & (s == NS - 1))
    def _():
        pltpu.sync_copy(xs_hbm.at[pl.ds(off, T_LAST)],
                        acc_sh.at[pl.ds(off, T_LAST)])
        pltpu.sync_copy(z2d_hbm.at[pl.ds(0, 16)],
                        acc_sh.at[pl.ds(N, 16)])

    @pl.when((c == 0) & (s != NS - 1))
    def _():
        pltpu.sync_copy(xs_hbm.at[pl.ds(off, T_CHUNK)],
                        acc_sh.at[pl.ds(off, T_CHUNK)])

    @pl.when((c == 1) & (s == NS - 1))
    def _():
        pltpu.sync_copy(z2d_hbm.at[pl.ds(0, T_LAST + 16)],
                        acc_sh.at[pl.ds(off, T_LAST + 16)])

    @pl.when((c == 1) & (s != NS - 1))
    def _():
        pltpu.sync_copy(z2d_hbm.at[pl.ds(0, T_CHUNK)],
                        acc_sh.at[pl.ds(off, T_CHUNK)])

    plsc.subcore_barrier()

    erow0 = g * RW

    def body(i, _):
        base = erow0 + i * CHUNK
        pltpu.sync_copy(s2d.at[pl.ds(base, CHUNK)], sidx_v)
        pltpu.sync_copy(r2d.at[pl.ds(base, CHUNK)], ridx_v)
        for h in range(CHUNK // RB):
            cps = [pltpu.async_copy(xs_hbm.at[sidx_v.at[h * RB + j]],
                                    rows_v.at[j], gsem)
                   for j in range(RB)]
            for cp in cps:
                cp.wait()
            scps = [pltpu.async_copy(rows_v.at[j],
                                     acc_sh.at[ridx_v.at[h * RB + j]],
                                     ssem, add=True)
                    for j in range(RB)]
            for cp in scps:
                cp.wait()
        return 0

    lax.fori_loop(0, RW // CHUNK, body, 0)
    plsc.subcore_barrier()

    @pl.when((c == 0) & (s == NS - 1))
    def _():
        pltpu.sync_copy(acc_sh.at[pl.ds(off, T_LAST)],
                        out_a.at[pl.ds(off, T_LAST)])

    @pl.when((c == 0) & (s != NS - 1))
    def _():
        pltpu.sync_copy(acc_sh.at[pl.ds(off, T_CHUNK)],
                        out_a.at[pl.ds(off, T_CHUNK)])

    @pl.when((c == 1) & (s == NS - 1))
    def _():
        pltpu.sync_copy(acc_sh.at[pl.ds(off, T_LAST)],
                        out_b.at[pl.ds(off, T_LAST)])

    @pl.when((c == 1) & (s != NS - 1))
    def _():
        pltpu.sync_copy(acc_sh.at[pl.ds(off, T_CHUNK)],
                        out_b.at[pl.ds(off, T_CHUNK)])


def _add_body(a_ref, b_ref, out_ref):
    out_ref[...] = a_ref[...] + b_ref[...]


def _add_tc(a, b):
    BR = 1000
    return pl.pallas_call(
        _add_body,
        grid=(N // BR,),
        in_specs=[pl.BlockSpec((BR, D), lambda i: (i, 0)),
                  pl.BlockSpec((BR, D), lambda i: (i, 0))],
        out_specs=pl.BlockSpec((BR, D), lambda i: (i, 0)),
        out_shape=jax.ShapeDtypeStruct((N, D), jnp.float32),
    )(a, b)


def kernel(nodes, senders, receivers, W):
    pad_bin = jnp.full((EPAD,), N, jnp.int32)
    pad_zero = jnp.zeros((EPAD,), jnp.int32)
    s2dh = jnp.concatenate([senders, pad_bin]).reshape(EROWS, 128)
    r2dh = jnp.concatenate([receivers, pad_bin]).reshape(EROWS, 128)
    s2dm = jnp.concatenate([senders, pad_zero]).reshape(EROWS, 128)
    r2dm = jnp.concatenate([receivers, pad_bin]).reshape(EROWS, 128)

    ones128 = jnp.ones((128,), jnp.float32)
    z1d = jnp.zeros((HT,), jnp.float32)
    z2d = jnp.zeros((T_LAST + 16, D), jnp.float32)

    hs_a, hr_a, hs_b, hr_b = _hist_sc(s2dh, r2dh, ones128, z1d)
    hs0 = hs_a[:N].reshape(N, 1)
    hs1 = hs_b[:N].reshape(N, 1)
    hr0 = hr_a[:N].reshape(N, 1)
    hr1 = hr_b[:N].reshape(N, 1)

    xs = _matmul_scale_tc(nodes, W, hs0, hs1, hr0, hr1)
    pa, pb = _message_sc(xs, s2dm, r2dm, z2d)
    return _add_tc(pa, pb)


# speedup vs baseline: 12.1747x; 2.5151x over previous
import functools

import jax
import jax.numpy as jnp
from jax import lax
from jax.experimental import pallas as pl
from jax.experimental.pallas import tpu as pltpu
from jax.experimental.pallas import tpu_sc as plsc

N = 10000
E = 320000
D = 128

NC = 2
NS = 16
NW = NC * NS

EROWS = 2560
EPAD = EROWS * 128 - E
RW = EROWS // NW
CHUNK = 8
RB = 2

NB = N + 64
HB = 10240
HT = HB // NS

T_CHUNK = 624
T_LAST = N - 15 * T_CHUNK

_mesh = plsc.VectorSubcoreMesh(core_axis_name="c", subcore_axis_name="s",
                               num_cores=NC, num_subcores=NS)


@functools.partial(
    pl.kernel,
    out_type=(jax.ShapeDtypeStruct((HB,), jnp.float32),
              jax.ShapeDtypeStruct((HB,), jnp.float32),
              jax.ShapeDtypeStruct((HB,), jnp.float32),
              jax.ShapeDtypeStruct((HB,), jnp.float32)),
    mesh=_mesh,
    scratch_types=[
        pltpu.VMEM((CHUNK, 128), jnp.int32),
        pltpu.VMEM((CHUNK, 128), jnp.int32),
        pltpu.VMEM((128,), jnp.float32),
        pltpu.VMEM_SHARED((HB,), jnp.float32),
        pltpu.VMEM_SHARED((HB,), jnp.float32),
        pltpu.SemaphoreType.DMA,
    ],
)
def _hist_sc(s2d, r2d, ones_hbm, z1d_hbm, hs_a, hr_a, hs_b, hr_b,
             sidx_v, ridx_v, ones_v, hs_sh, hr_sh, sem):
    c = lax.axis_index("c")
    s = lax.axis_index("s")
    g = s * NC + c

    off = s * HT
    pltpu.sync_copy(z1d_hbm, hs_sh.at[pl.ds(off, HT)])
    pltpu.sync_copy(z1d_hbm, hr_sh.at[pl.ds(off, HT)])
    pltpu.sync_copy(ones_hbm, ones_v)
    plsc.subcore_barrier()

    row0 = g * RW

    def body(i, _):
        base = row0 + i * CHUNK
        pltpu.sync_copy(s2d.at[pl.ds(base, CHUNK)], sidx_v)
        pltpu.sync_copy(r2d.at[pl.ds(base, CHUNK)], ridx_v)
        for j in range(CHUNK):
            pltpu.sync_copy(ones_v, hs_sh.at[sidx_v.at[j]], add=True)
            pltpu.sync_copy(ones_v, hr_sh.at[ridx_v.at[j]], add=True)
        return 0

    lax.fori_loop(0, RW // CHUNK, body, 0)
    plsc.subcore_barrier()

    @pl.when(c == 0)
    def _():
        pltpu.sync_copy(hs_sh.at[pl.ds(off, HT)], hs_a.at[pl.ds(off, HT)])
        pltpu.sync_copy(hr_sh.at[pl.ds(off, HT)], hr_a.at[pl.ds(off, HT)])

    @pl.when(c == 1)
    def _():
        pltpu.sync_copy(hs_sh.at[pl.ds(off, HT)], hs_b.at[pl.ds(off, HT)])
        pltpu.sync_copy(hr_sh.at[pl.ds(off, HT)], hr_b.at[pl.ds(off, HT)])


def _matmul_scale_body(nodes_ref, w_ref, hs0_ref, hs1_ref, hr0_ref, hr1_ref,
                       out_ref):
    cs = hs0_ref[...] + hs1_ref[...]
    cr = hr0_ref[...] + hr1_ref[...]
    scale = lax.rsqrt((2.0 * cs + 2.0) * (2.0 * cr + 2.0))
    x = jnp.dot(nodes_ref[...], w_ref[...], preferred_element_type=jnp.float32)
    out_ref[...] = x * scale


def _matmul_scale_tc(nodes, W, hs0, hs1, hr0, hr1):
    BR = 1000
    grid = (N // BR,)
    return pl.pallas_call(
        _matmul_scale_body,
        grid=grid,
        in_specs=[
            pl.BlockSpec((BR, D), lambda i: (i, 0)),
            pl.BlockSpec((D, D), lambda i: (0, 0)),
            pl.BlockSpec((BR, 1), lambda i: (i, 0)),
            pl.BlockSpec((BR, 1), lambda i: (i, 0)),
            pl.BlockSpec((BR, 1), lambda i: (i, 0)),
            pl.BlockSpec((BR, 1), lambda i: (i, 0)),
        ],
        out_specs=pl.BlockSpec((BR, D), lambda i: (i, 0)),
        out_shape=jax.ShapeDtypeStruct((N, D), jnp.float32),
    )(nodes, W, hs0, hs1, hr0, hr1)


@functools.partial(
    pl.kernel,
    out_type=(jax.ShapeDtypeStruct((N, D), jnp.float32),
              jax.ShapeDtypeStruct((N, D), jnp.float32)),
    mesh=_mesh,
    scratch_types=[
        pltpu.VMEM((CHUNK, 128), jnp.int32),
        pltpu.VMEM((CHUNK, 128), jnp.int32),
        pltpu.VMEM((RB, 128, D), jnp.float32),
        pltpu.VMEM_SHARED((NB, D), jnp.float32),
        pltpu.SemaphoreType.DMA,
        pltpu.SemaphoreType.DMA,
    ],
)
def _message_sc(xs_hbm, s2d, r2d, z2d_hbm, out_a, out_b,
                sidx_v, ridx_v, rows_v, acc_sh, gsem, ssem):
    c = lax.axis_index("c")
    s = lax.axis_index("s")
    g = s * NC + c

    off = s * T_CHUNK

    @pl.when((c == 0) & (s == NS - 1))
    def _():
        pltpu.sync_copy(xs_hbm.at[pl.ds(off, T_LAST)],
                        acc_sh.at[pl.ds(off, T_LAST)])

    @pl.when((c == 0) & (s != NS - 1))
    def _():
        pltpu.sync_copy(xs_hbm.at[pl.ds(off, T_CHUNK)],
                        acc_sh.at[pl.ds(off, T_CHUNK)])

    @pl.when((c == 1) & (s == NS - 1))
    def _():
        pltpu.sync_copy(z2d_hbm.at[pl.ds(0, T_LAST)],
                        acc_sh.at[pl.ds(off, T_LAST)])

    @pl.when((c == 1) & (s != NS - 1))
    def _():
        pltpu.sync_copy(z2d_hbm.at[pl.ds(0, T_CHUNK)],
                        acc_sh.at[pl.ds(off, T_CHUNK)])

    plsc.subcore_barrier()

    erow0 = g * RW

    def body(i, _):
        base = erow0 + i * CHUNK
        pltpu.sync_copy(s2d.at[pl.ds(base, CHUNK)], sidx_v)
        pltpu.sync_copy(r2d.at[pl.ds(base, CHUNK)], ridx_v)
        for h in range(CHUNK // RB):
            cps = [pltpu.async_copy(xs_hbm.at[sidx_v.at[h * RB + j]],
                                    rows_v.at[j], gsem)
                   for j in range(RB)]
            for cp in cps:
                cp.wait()
            scps = [pltpu.async_copy(rows_v.at[j],
                                     acc_sh.at[ridx_v.at[h * RB + j]],
                                     ssem, add=True)
                    for j in range(RB)]
            for cp in scps:
                cp.wait()
        return 0

    lax.fori_loop(0, RW // CHUNK, body, 0)
    plsc.subcore_barrier()

    @pl.when((c == 0) & (s == NS - 1))
    def _():
        pltpu.sync_copy(acc_sh.at[pl.ds(off, T_LAST)],
                        out_a.at[pl.ds(off, T_LAST)])

    @pl.when((c == 0) & (s != NS - 1))
    def _():
        pltpu.sync_copy(acc_sh.at[pl.ds(off, T_CHUNK)],
                        out_a.at[pl.ds(off, T_CHUNK)])

    @pl.when((c == 1) & (s == NS - 1))
    def _():
        pltpu.sync_copy(acc_sh.at[pl.ds(off, T_LAST)],
                        out_b.at[pl.ds(off, T_LAST)])

    @pl.when((c == 1) & (s != NS - 1))
    def _():
        pltpu.sync_copy(acc_sh.at[pl.ds(off, T_CHUNK)],
                        out_b.at[pl.ds(off, T_CHUNK)])


def _add_body(a_ref, b_ref, out_ref):
    out_ref[...] = a_ref[...] + b_ref[...]


def _add_tc(a, b):
    BR = 1000
    return pl.pallas_call(
        _add_body,
        grid=(N // BR,),
        in_specs=[pl.BlockSpec((BR, D), lambda i: (i, 0)),
                  pl.BlockSpec((BR, D), lambda i: (i, 0))],
        out_specs=pl.BlockSpec((BR, D), lambda i: (i, 0)),
        out_shape=jax.ShapeDtypeStruct((N, D), jnp.float32),
    )(a, b)


def kernel(nodes, senders, receivers, W):
    ar = jnp.arange(EPAD, dtype=jnp.int32)
    pad_hist = N + (ar % 240)
    pad_recv = N + (ar % 64)
    pad_send = ar % 8192

    def interleave(x, pad):
        xw = x.reshape(NW, E // NW)
        pw = pad.reshape(NW, EPAD // NW)
        return jnp.concatenate([xw, pw], axis=1).reshape(EROWS, 128)

    s2dh = interleave(senders, pad_hist)
    r2dh = interleave(receivers, pad_hist)
    s2dm = interleave(senders, pad_send)
    r2dm = interleave(receivers, pad_recv)

    ones128 = jnp.ones((128,), jnp.float32)
    z1d = jnp.zeros((HT,), jnp.float32)
    z2d = jnp.zeros((T_LAST, D), jnp.float32)

    hs_a, hr_a, hs_b, hr_b = _hist_sc(s2dh, r2dh, ones128, z1d)
    hs0 = hs_a[:N].reshape(N, 1)
    hs1 = hs_b[:N].reshape(N, 1)
    hr0 = hr_a[:N].reshape(N, 1)
    hr1 = hr_b[:N].reshape(N, 1)

    xs = _matmul_scale_tc(nodes, W, hs0, hs1, hr0, hr1)
    pa, pb = _message_sc(xs, s2dm, r2dm, z2d)
    return _add_tc(pa, pb)


# speedup vs baseline: 13.5865x; 1.1160x over previous
import functools

import jax
import jax.numpy as jnp
from jax import lax
from jax.experimental import pallas as pl
from jax.experimental.pallas import tpu as pltpu
from jax.experimental.pallas import tpu_sc as plsc

N = 10000
E = 320000
D = 128

NC = 2
NS = 16
NW = NC * NS

EROWS = 2560
EPAD = EROWS * 128 - E
RW = EROWS // NW
CHUNK = 8
RB = 2
HLF = RW // 2

NB = N + 64
HB = 10240
HT = HB // NS

T_CHUNK = 624
T_LAST = N - 15 * T_CHUNK

_mesh = plsc.VectorSubcoreMesh(core_axis_name="c", subcore_axis_name="s",
                               num_cores=NC, num_subcores=NS)


@functools.partial(
    pl.kernel,
    out_type=(jax.ShapeDtypeStruct((HB,), jnp.float32),
              jax.ShapeDtypeStruct((HB,), jnp.float32),
              jax.ShapeDtypeStruct((HB,), jnp.float32),
              jax.ShapeDtypeStruct((HB,), jnp.float32)),
    mesh=_mesh,
    scratch_types=[
        pltpu.VMEM((CHUNK, 128), jnp.int32),
        pltpu.VMEM((CHUNK, 128), jnp.int32),
        pltpu.VMEM((128,), jnp.float32),
        pltpu.VMEM_SHARED((HB,), jnp.float32),
        pltpu.VMEM_SHARED((HB,), jnp.float32),
        pltpu.SemaphoreType.DMA,
    ],
)
def _hist_sc(s2d, r2d, ones_hbm, z1d_hbm, hs_a, hr_a, hs_b, hr_b,
             sidx_v, ridx_v, ones_v, hs_sh, hr_sh, sem):
    c = lax.axis_index("c")
    s = lax.axis_index("s")
    g = s * NC + c

    off = s * HT
    pltpu.sync_copy(z1d_hbm, hs_sh.at[pl.ds(off, HT)])
    pltpu.sync_copy(z1d_hbm, hr_sh.at[pl.ds(off, HT)])
    pltpu.sync_copy(ones_hbm, ones_v)
    plsc.subcore_barrier()

    row0 = g * RW

    def body(i, _):
        base = row0 + i * CHUNK
        pltpu.sync_copy(s2d.at[pl.ds(base, CHUNK)], sidx_v)
        pltpu.sync_copy(r2d.at[pl.ds(base, CHUNK)], ridx_v)
        for j in range(CHUNK):
            pltpu.sync_copy(ones_v, hs_sh.at[sidx_v.at[j]], add=True)
            pltpu.sync_copy(ones_v, hr_sh.at[ridx_v.at[j]], add=True)
        return 0

    lax.fori_loop(0, RW // CHUNK, body, 0)
    plsc.subcore_barrier()

    @pl.when(c == 0)
    def _():
        pltpu.sync_copy(hs_sh.at[pl.ds(off, HT)], hs_a.at[pl.ds(off, HT)])
        pltpu.sync_copy(hr_sh.at[pl.ds(off, HT)], hr_a.at[pl.ds(off, HT)])

    @pl.when(c == 1)
    def _():
        pltpu.sync_copy(hs_sh.at[pl.ds(off, HT)], hs_b.at[pl.ds(off, HT)])
        pltpu.sync_copy(hr_sh.at[pl.ds(off, HT)], hr_b.at[pl.ds(off, HT)])


def _matmul_scale_body(nodes_ref, w_ref, hs0_ref, hs1_ref, hr0_ref, hr1_ref,
                       out_ref):
    cs = hs0_ref[...] + hs1_ref[...]
    cr = hr0_ref[...] + hr1_ref[...]
    scale = lax.rsqrt((2.0 * cs + 2.0) * (2.0 * cr + 2.0))
    x = jnp.dot(nodes_ref[...], w_ref[...], preferred_element_type=jnp.float32)
    out_ref[...] = x * scale


def _matmul_scale_tc(nodes, W, hs0, hs1, hr0, hr1):
    BR = 1000
    grid = (N // BR,)
    return pl.pallas_call(
        _matmul_scale_body,
        grid=grid,
        in_specs=[
            pl.BlockSpec((BR, D), lambda i: (i, 0)),
            pl.BlockSpec((D, D), lambda i: (0, 0)),
            pl.BlockSpec((BR, 1), lambda i: (i, 0)),
            pl.BlockSpec((BR, 1), lambda i: (i, 0)),
            pl.BlockSpec((BR, 1), lambda i: (i, 0)),
            pl.BlockSpec((BR, 1), lambda i: (i, 0)),
        ],
        out_specs=pl.BlockSpec((BR, D), lambda i: (i, 0)),
        out_shape=jax.ShapeDtypeStruct((N, D), jnp.float32),
    )(nodes, W, hs0, hs1, hr0, hr1)


@functools.partial(
    pl.kernel,
    out_type=(jax.ShapeDtypeStruct((N, D), jnp.float32),
              jax.ShapeDtypeStruct((N, D), jnp.float32)),
    mesh=_mesh,
    scratch_types=[
        pltpu.VMEM((HLF, 128), jnp.int32),
        pltpu.VMEM((HLF, 128), jnp.int32),
        pltpu.VMEM((RB, 128, D), jnp.float32),
        pltpu.VMEM_SHARED((NB, D), jnp.float32),
        pltpu.SemaphoreType.DMA,
        pltpu.SemaphoreType.DMA,
        pltpu.SemaphoreType.DMA,
        pltpu.SemaphoreType.DMA,
    ],
)
def _message_sc(xs_hbm, s2d, r2d, z2d_hbm, out_a, out_b,
                sidx_v, ridx_v, rows_v, acc_sh, gsem0, gsem1, ssem0, ssem1):
    c = lax.axis_index("c")
    s = lax.axis_index("s")
    g = s * NC + c

    off = s * T_CHUNK

    @pl.when((c == 0) & (s == NS - 1))
    def _():
        pltpu.sync_copy(xs_hbm.at[pl.ds(off, T_LAST)],
                        acc_sh.at[pl.ds(off, T_LAST)])

    @pl.when((c == 0) & (s != NS - 1))
    def _():
        pltpu.sync_copy(xs_hbm.at[pl.ds(off, T_CHUNK)],
                        acc_sh.at[pl.ds(off, T_CHUNK)])

    @pl.when((c == 1) & (s == NS - 1))
    def _():
        pltpu.sync_copy(z2d_hbm.at[pl.ds(0, T_LAST)],
                        acc_sh.at[pl.ds(off, T_LAST)])

    @pl.when((c == 1) & (s != NS - 1))
    def _():
        pltpu.sync_copy(z2d_hbm.at[pl.ds(0, T_CHUNK)],
                        acc_sh.at[pl.ds(off, T_CHUNK)])

    plsc.subcore_barrier()

    erow0 = g * RW
    gsems = (gsem0, gsem1)
    ssems = (ssem0, ssem1)

    def drain(sem, k):
        pltpu.make_async_copy(xs_hbm.at[pl.ds(0, 128)],
                              rows_v.at[k], sem).wait()

    for half in range(2):
        hbase = erow0 + half * HLF
        pltpu.sync_copy(s2d.at[pl.ds(hbase, HLF)], sidx_v)
        pltpu.sync_copy(r2d.at[pl.ds(hbase, HLF)], ridx_v)
        pltpu.async_copy(xs_hbm.at[sidx_v.at[0]], rows_v.at[0], gsem0)

        def ring(i, _):
            for j in range(8):
                r = i * 8 + j
                k = j % 2
                drain(gsems[k], k)
                pltpu.async_copy(rows_v.at[k], acc_sh.at[ridx_v.at[r]],
                                 ssems[k], add=True)

                @pl.when(r + 1 < HLF)
                def _():
                    @pl.when(r >= 1)
                    def _():
                        drain(ssems[1 - k], 1 - k)

                    pltpu.async_copy(xs_hbm.at[sidx_v.at[r + 1]],
                                     rows_v.at[1 - k], gsems[1 - k])
            return 0

        lax.fori_loop(0, HLF // 8, ring, 0)
        drain(ssems[0], 0)
        drain(ssems[1], 1)

    plsc.subcore_barrier()

    @pl.when((c == 0) & (s == NS - 1))
    def _():
        pltpu.sync_copy(acc_sh.at[pl.ds(off, T_LAST)],
                        out_a.at[pl.ds(off, T_LAST)])

    @pl.when((c == 0) & (s != NS - 1))
    def _():
        pltpu.sync_copy(acc_sh.at[pl.ds(off, T_CHUNK)],
                        out_a.at[pl.ds(off, T_CHUNK)])

    @pl.when((c == 1) & (s == NS - 1))
    def _():
        pltpu.sync_copy(acc_sh.at[pl.ds(off, T_LAST)],
                        out_b.at[pl.ds(off, T_LAST)])

    @pl.when((c == 1) & (s != NS - 1))
    def _():
        pltpu.sync_copy(acc_sh.at[pl.ds(off, T_CHUNK)],
                        out_b.at[pl.ds(off, T_CHUNK)])


def _add_body(a_ref, b_ref, out_ref):
    out_ref[...] = a_ref[...] + b_ref[...]


def _add_tc(a, b):
    BR = 1000
    return pl.pallas_call(
        _add_body,
        grid=(N // BR,),
        in_specs=[pl.BlockSpec((BR, D), lambda i: (i, 0)),
                  pl.BlockSpec((BR, D), lambda i: (i, 0))],
        out_specs=pl.BlockSpec((BR, D), lambda i: (i, 0)),
        out_shape=jax.ShapeDtypeStruct((N, D), jnp.float32),
    )(a, b)


def kernel(nodes, senders, receivers, W):
    ar = jnp.arange(EPAD, dtype=jnp.int32)
    pad_hist = N + (ar % 240)
    pad_recv = N + (ar % 64)
    pad_send = ar % 8192

    def interleave(x, pad):
        xw = x.reshape(NW, E // NW)
        pw = pad.reshape(NW, EPAD // NW)
        return jnp.concatenate([xw, pw], axis=1).reshape(EROWS, 128)

    s2dh = interleave(senders, pad_hist)
    r2dh = interleave(receivers, pad_hist)
    s2dm = interleave(senders, pad_send)
    r2dm = interleave(receivers, pad_recv)

    ones128 = jnp.ones((128,), jnp.float32)
    z1d = jnp.zeros((HT,), jnp.float32)
    z2d = jnp.zeros((T_LAST, D), jnp.float32)

    hs_a, hr_a, hs_b, hr_b = _hist_sc(s2dh, r2dh, ones128, z1d)
    hs0 = hs_a[:N].reshape(N, 1)
    hs1 = hs_b[:N].reshape(N, 1)
    hr0 = hr_a[:N].reshape(N, 1)
    hr1 = hr_b[:N].reshape(N, 1)

    xs = _matmul_scale_tc(nodes, W, hs0, hs1, hr0, hr1)
    pa, pb = _message_sc(xs, s2dm, r2dm, z2d)
    return _add_tc(pa, pb)


# speedup vs baseline: 14.2858x; 1.0515x over previous
import functools

import jax
import jax.numpy as jnp
from jax import lax
from jax.experimental import pallas as pl
from jax.experimental.pallas import tpu as pltpu
from jax.experimental.pallas import tpu_sc as plsc

N = 10000
E = 320000
D = 128

NC = 2
NS = 16
NW = NC * NS

EROWS = 2560
EPAD = EROWS * 128 - E
RW = EROWS // NW
CHUNK = 8
RB = 2
HLF = RW // 2

NB = N + 64
HB = 10240
HT = HB // NS

T_CHUNK = 624
T_LAST = N - 15 * T_CHUNK

_mesh = plsc.VectorSubcoreMesh(core_axis_name="c", subcore_axis_name="s",
                               num_cores=NC, num_subcores=NS)


@functools.partial(
    pl.kernel,
    out_type=(jax.ShapeDtypeStruct((HB,), jnp.float32),
              jax.ShapeDtypeStruct((HB,), jnp.float32),
              jax.ShapeDtypeStruct((HB,), jnp.float32),
              jax.ShapeDtypeStruct((HB,), jnp.float32)),
    mesh=_mesh,
    scratch_types=[
        pltpu.VMEM((CHUNK, 128), jnp.int32),
        pltpu.VMEM((CHUNK, 128), jnp.int32),
        pltpu.VMEM((128,), jnp.float32),
        pltpu.VMEM_SHARED((HB,), jnp.float32),
        pltpu.VMEM_SHARED((HB,), jnp.float32),
        pltpu.SemaphoreType.DMA,
    ],
)
def _hist_sc(s2d, r2d, ones_hbm, z1d_hbm, hs_a, hr_a, hs_b, hr_b,
             sidx_v, ridx_v, ones_v, hs_sh, hr_sh, sem):
    c = lax.axis_index("c")
    s = lax.axis_index("s")
    g = s * NC + c

    off = s * HT
    pltpu.sync_copy(z1d_hbm, hs_sh.at[pl.ds(off, HT)])
    pltpu.sync_copy(z1d_hbm, hr_sh.at[pl.ds(off, HT)])
    pltpu.sync_copy(ones_hbm, ones_v)
    plsc.subcore_barrier()

    row0 = g * RW

    def body(i, _):
        base = row0 + i * CHUNK
        pltpu.sync_copy(s2d.at[pl.ds(base, CHUNK)], sidx_v)
        pltpu.sync_copy(r2d.at[pl.ds(base, CHUNK)], ridx_v)
        cps = [pltpu.async_copy(ones_v, hs_sh.at[sidx_v.at[j]], sem, add=True)
               for j in range(CHUNK)]
        cps += [pltpu.async_copy(ones_v, hr_sh.at[ridx_v.at[j]], sem, add=True)
                for j in range(CHUNK)]
        for cp in cps:
            cp.wait()
        return 0

    lax.fori_loop(0, RW // CHUNK, body, 0)
    plsc.subcore_barrier()

    @pl.when(c == 0)
    def _():
        pltpu.sync_copy(hs_sh.at[pl.ds(off, HT)], hs_a.at[pl.ds(off, HT)])
        pltpu.sync_copy(hr_sh.at[pl.ds(off, HT)], hr_a.at[pl.ds(off, HT)])

    @pl.when(c == 1)
    def _():
        pltpu.sync_copy(hs_sh.at[pl.ds(off, HT)], hs_b.at[pl.ds(off, HT)])
        pltpu.sync_copy(hr_sh.at[pl.ds(off, HT)], hr_b.at[pl.ds(off, HT)])


def _matmul_body(nodes_ref, w_ref, out_ref):
    out_ref[...] = jnp.dot(nodes_ref[...], w_ref[...],
                           preferred_element_type=jnp.float32)


def _matmul_tc(nodes, W):
    BR = 1000
    return pl.pallas_call(
        _matmul_body,
        grid=(N // BR,),
        in_specs=[pl.BlockSpec((BR, D), lambda i: (i, 0)),
                  pl.BlockSpec((D, D), lambda i: (0, 0))],
        out_specs=pl.BlockSpec((BR, D), lambda i: (i, 0)),
        out_shape=jax.ShapeDtypeStruct((N, D), jnp.float32),
    )(nodes, W)


def _scale_body(x_ref, hs0_ref, hs1_ref, hr0_ref, hr1_ref, out_ref):
    cs = hs0_ref[...] + hs1_ref[...]
    cr = hr0_ref[...] + hr1_ref[...]
    scale = lax.rsqrt((2.0 * cs + 2.0) * (2.0 * cr + 2.0))
    out_ref[...] = x_ref[...] * scale


def _scale_tc(x, hs0, hs1, hr0, hr1):
    BR = 1000
    return pl.pallas_call(
        _scale_body,
        grid=(N // BR,),
        in_specs=[
            pl.BlockSpec((BR, D), lambda i: (i, 0)),
            pl.BlockSpec((BR, 1), lambda i: (i, 0)),
            pl.BlockSpec((BR, 1), lambda i: (i, 0)),
            pl.BlockSpec((BR, 1), lambda i: (i, 0)),
            pl.BlockSpec((BR, 1), lambda i: (i, 0)),
        ],
        out_specs=pl.BlockSpec((BR, D), lambda i: (i, 0)),
        out_shape=jax.ShapeDtypeStruct((N, D), jnp.float32),
    )(x, hs0, hs1, hr0, hr1)


@functools.partial(
    pl.kernel,
    out_type=(jax.ShapeDtypeStruct((N, D), jnp.float32),
              jax.ShapeDtypeStruct((N, D), jnp.float32)),
    mesh=_mesh,
    scratch_types=[
        pltpu.VMEM((HLF, 128), jnp.int32),
        pltpu.VMEM((HLF, 128), jnp.int32),
        pltpu.VMEM((RB, 128, D), jnp.float32),
        pltpu.VMEM_SHARED((NB, D), jnp.float32),
        pltpu.SemaphoreType.DMA,
        pltpu.SemaphoreType.DMA,
        pltpu.SemaphoreType.DMA,
        pltpu.SemaphoreType.DMA,
    ],
)
def _message_sc(xs_hbm, s2d, r2d, z2d_hbm, out_a, out_b,
                sidx_v, ridx_v, rows_v, acc_sh, gsem0, gsem1, ssem0, ssem1):
    c = lax.axis_index("c")
    s = lax.axis_index("s")
    g = s * NC + c

    off = s * T_CHUNK

    @pl.when((c == 0) & (s == NS - 1))
    def _():
        pltpu.sync_copy(xs_hbm.at[pl.ds(off, T_LAST)],
                        acc_sh.at[pl.ds(off, T_LAST)])

    @pl.when((c == 0) & (s != NS - 1))
    def _():
        pltpu.sync_copy(xs_hbm.at[pl.ds(off, T_CHUNK)],
                        acc_sh.at[pl.ds(off, T_CHUNK)])

    @pl.when((c == 1) & (s == NS - 1))
    def _():
        pltpu.sync_copy(z2d_hbm.at[pl.ds(0, T_LAST)],
                        acc_sh.at[pl.ds(off, T_LAST)])

    @pl.when((c == 1) & (s != NS - 1))
    def _():
        pltpu.sync_copy(z2d_hbm.at[pl.ds(0, T_CHUNK)],
                        acc_sh.at[pl.ds(off, T_CHUNK)])

    plsc.subcore_barrier()

    erow0 = g * RW
    gsems = (gsem0, gsem1)
    ssems = (ssem0, ssem1)

    def drain(sem, k):
        pltpu.make_async_copy(xs_hbm.at[pl.ds(0, 128)],
                              rows_v.at[k], sem).wait()

    for half in range(2):
        hbase = erow0 + half * HLF
        pltpu.sync_copy(s2d.at[pl.ds(hbase, HLF)], sidx_v)
        pltpu.sync_copy(r2d.at[pl.ds(hbase, HLF)], ridx_v)
        pltpu.async_copy(xs_hbm.at[sidx_v.at[0]], rows_v.at[0], gsem0)

        def ring(i, _):
            for j in range(8):
                r = i * 8 + j
                k = j % 2
                drain(gsems[k], k)
                pltpu.async_copy(rows_v.at[k], acc_sh.at[ridx_v.at[r]],
                                 ssems[k], add=True)

                @pl.when(r + 1 < HLF)
                def _():
                    @pl.when(r >= 1)
                    def _():
                        drain(ssems[1 - k], 1 - k)

                    pltpu.async_copy(xs_hbm.at[sidx_v.at[r + 1]],
                                     rows_v.at[1 - k], gsems[1 - k])
            return 0

        lax.fori_loop(0, HLF // 8, ring, 0)
        drain(ssems[0], 0)
        drain(ssems[1], 1)

    plsc.subcore_barrier()

    @pl.when((c == 0) & (s == NS - 1))
    def _():
        pltpu.sync_copy(acc_sh.at[pl.ds(off, T_LAST)],
                        out_a.at[pl.ds(off, T_LAST)])

    @pl.when((c == 0) & (s != NS - 1))
    def _():
        pltpu.sync_copy(acc_sh.at[pl.ds(off, T_CHUNK)],
                        out_a.at[pl.ds(off, T_CHUNK)])

    @pl.when((c == 1) & (s == NS - 1))
    def _():
        pltpu.sync_copy(acc_sh.at[pl.ds(off, T_LAST)],
                        out_b.at[pl.ds(off, T_LAST)])

    @pl.when((c == 1) & (s != NS - 1))
    def _():
        pltpu.sync_copy(acc_sh.at[pl.ds(off, T_CHUNK)],
                        out_b.at[pl.ds(off, T_CHUNK)])


def _add_body(a_ref, b_ref, out_ref):
    out_ref[...] = a_ref[...] + b_ref[...]


def _add_tc(a, b):
    BR = 1000
    return pl.pallas_call(
        _add_body,
        grid=(N // BR,),
        in_specs=[pl.BlockSpec((BR, D), lambda i: (i, 0)),
                  pl.BlockSpec((BR, D), lambda i: (i, 0))],
        out_specs=pl.BlockSpec((BR, D), lambda i: (i, 0)),
        out_shape=jax.ShapeDtypeStruct((N, D), jnp.float32),
    )(a, b)


def kernel(nodes, senders, receivers, W):
    ar = jnp.arange(EPAD, dtype=jnp.int32)
    pad_trash = N + (ar % 64)
    pad_send = ar % 8192

    def interleave(x, pad):
        xw = x.reshape(NW, E // NW)
        pw = pad.reshape(NW, EPAD // NW)
        return jnp.concatenate([xw, pw], axis=1).reshape(EROWS, 128)

    s2dh = interleave(senders, pad_trash)
    s2dm = interleave(senders, pad_send)
    r2dp = interleave(receivers, pad_trash)

    ones128 = jnp.ones((128,), jnp.float32)
    z1d = jnp.zeros((HT,), jnp.float32)
    z2d = jnp.zeros((T_LAST, D), jnp.float32)

    x = _matmul_tc(nodes, W)
    hs_a, hr_a, hs_b, hr_b = _hist_sc(s2dh, r2dp, ones128, z1d)
    hs0 = hs_a[:N].reshape(N, 1)
    hs1 = hs_b[:N].reshape(N, 1)
    hr0 = hr_a[:N].reshape(N, 1)
    hr1 = hr_b[:N].reshape(N, 1)

    xs = _scale_tc(x, hs0, hs1, hr0, hr1)
    pa, pb = _message_sc(xs, s2dm, r2dp, z2d)
    return _add_tc(pa, pb)


# speedup vs baseline: 15.7009x; 1.0991x over previous
import functools

import jax
import jax.numpy as jnp
from jax import lax
from jax.experimental import pallas as pl
from jax.experimental.pallas import tpu as pltpu
from jax.experimental.pallas import tpu_sc as plsc

N = 10000
E = 320000
D = 128

NC = 2
NS = 16
NW = NC * NS

EROWS = 2560
EPAD = EROWS * 128 - E
RW = EROWS // NW
CHUNK = 8
RB = 2
HLF = RW // 2

NB = N + 64
HB = 10240
HT = HB // NS

T_CHUNK = 624
T_LAST = N - 15 * T_CHUNK

_mesh = plsc.VectorSubcoreMesh(core_axis_name="c", subcore_axis_name="s",
                               num_cores=NC, num_subcores=NS)


@functools.partial(
    pl.kernel,
    out_type=(jax.ShapeDtypeStruct((HB,), jnp.float32),
              jax.ShapeDtypeStruct((HB,), jnp.float32),
              jax.ShapeDtypeStruct((HB,), jnp.float32),
              jax.ShapeDtypeStruct((HB,), jnp.float32)),
    mesh=_mesh,
    scratch_types=[
        pltpu.VMEM((CHUNK, 128), jnp.int32),
        pltpu.VMEM((CHUNK, 128), jnp.int32),
        pltpu.VMEM((128,), jnp.float32),
        pltpu.VMEM_SHARED((HB,), jnp.float32),
        pltpu.VMEM_SHARED((HB,), jnp.float32),
        pltpu.SemaphoreType.DMA,
    ],
)
def _hist_sc(s2d, r2d, ones_hbm, z1d_hbm, hs_a, hr_a, hs_b, hr_b,
             sidx_v, ridx_v, ones_v, hs_sh, hr_sh, sem):
    c = lax.axis_index("c")
    s = lax.axis_index("s")
    g = s * NC + c

    off = s * HT
    pltpu.sync_copy(z1d_hbm, hs_sh.at[pl.ds(off, HT)])
    pltpu.sync_copy(z1d_hbm, hr_sh.at[pl.ds(off, HT)])
    pltpu.sync_copy(ones_hbm, ones_v)
    plsc.subcore_barrier()

    row0 = g * RW

    def body(i, _):
        base = row0 + i * CHUNK
        pltpu.sync_copy(s2d.at[pl.ds(base, CHUNK)], sidx_v)
        pltpu.sync_copy(r2d.at[pl.ds(base, CHUNK)], ridx_v)
        cps = [pltpu.async_copy(ones_v, hs_sh.at[sidx_v.at[j]], sem, add=True)
               for j in range(CHUNK)]
        cps += [pltpu.async_copy(ones_v, hr_sh.at[ridx_v.at[j]], sem, add=True)
                for j in range(CHUNK)]
        for cp in cps:
            cp.wait()
        return 0

    lax.fori_loop(0, RW // CHUNK, body, 0)
    plsc.subcore_barrier()

    @pl.when(c == 0)
    def _():
        pltpu.sync_copy(hs_sh.at[pl.ds(off, HT)], hs_a.at[pl.ds(off, HT)])
        pltpu.sync_copy(hr_sh.at[pl.ds(off, HT)], hr_a.at[pl.ds(off, HT)])

    @pl.when(c == 1)
    def _():
        pltpu.sync_copy(hs_sh.at[pl.ds(off, HT)], hs_b.at[pl.ds(off, HT)])
        pltpu.sync_copy(hr_sh.at[pl.ds(off, HT)], hr_b.at[pl.ds(off, HT)])


def _matmul_body(nodes_ref, w_ref, out_ref):
    out_ref[...] = jnp.dot(nodes_ref[...], w_ref[...],
                           preferred_element_type=jnp.float32)


def _matmul_tc(nodes_pad, W):
    BR = 1024
    return pl.pallas_call(
        _matmul_body,
        grid=(HB // BR,),
        in_specs=[pl.BlockSpec((BR, D), lambda i: (i, 0)),
                  pl.BlockSpec((D, D), lambda i: (0, 0))],
        out_specs=pl.BlockSpec((BR, D), lambda i: (i, 0)),
        out_shape=jax.ShapeDtypeStruct((HB, D), jnp.float32),
    )(nodes_pad, W)


def _scale_body(x_ref, hs0_ref, hs1_ref, hr0_ref, hr1_ref, eye_ref, out_ref):
    cs = hs0_ref[...] + hs1_ref[...]
    cr = hr0_ref[...] + hr1_ref[...]
    scale = lax.rsqrt((2.0 * cs + 2.0) * (2.0 * cr + 2.0))
    for j in range(8):
        diag = eye_ref[...] * scale[j:j + 1, :]
        out_ref[pl.ds(j * 128, 128), :] = jnp.dot(
            diag, x_ref[pl.ds(j * 128, 128), :],
            preferred_element_type=jnp.float32,
            precision=lax.Precision.HIGHEST)


def _scale_tc(x, hs0, hs1, hr0, hr1, eye):
    BR = 1024
    return pl.pallas_call(
        _scale_body,
        grid=(HB // BR,),
        in_specs=[
            pl.BlockSpec((BR, D), lambda i: (i, 0)),
            pl.BlockSpec((8, 128), lambda i: (i, 0)),
            pl.BlockSpec((8, 128), lambda i: (i, 0)),
            pl.BlockSpec((8, 128), lambda i: (i, 0)),
            pl.BlockSpec((8, 128), lambda i: (i, 0)),
            pl.BlockSpec((128, 128), lambda i: (0, 0)),
        ],
        out_specs=pl.BlockSpec((BR, D), lambda i: (i, 0)),
        out_shape=jax.ShapeDtypeStruct((HB, D), jnp.float32),
    )(x, hs0, hs1, hr0, hr1, eye)


@functools.partial(
    pl.kernel,
    out_type=(jax.ShapeDtypeStruct((N, D), jnp.float32),
              jax.ShapeDtypeStruct((N, D), jnp.float32)),
    mesh=_mesh,
    scratch_types=[
        pltpu.VMEM((HLF, 128), jnp.int32),
        pltpu.VMEM((HLF, 128), jnp.int32),
        pltpu.VMEM((RB, 128, D), jnp.float32),
        pltpu.VMEM_SHARED((NB, D), jnp.float32),
        pltpu.SemaphoreType.DMA,
        pltpu.SemaphoreType.DMA,
        pltpu.SemaphoreType.DMA,
        pltpu.SemaphoreType.DMA,
    ],
)
def _message_sc(xs_hbm, s2d, r2d, z2d_hbm, out_a, out_b,
                sidx_v, ridx_v, rows_v, acc_sh, gsem0, gsem1, ssem0, ssem1):
    c = lax.axis_index("c")
    s = lax.axis_index("s")
    g = s * NC + c

    off = s * T_CHUNK

    @pl.when((c == 0) & (s == NS - 1))
    def _():
        pltpu.sync_copy(xs_hbm.at[pl.ds(off, T_LAST)],
                        acc_sh.at[pl.ds(off, T_LAST)])

    @pl.when((c == 0) & (s != NS - 1))
    def _():
        pltpu.sync_copy(xs_hbm.at[pl.ds(off, T_CHUNK)],
                        acc_sh.at[pl.ds(off, T_CHUNK)])

    @pl.when((c == 1) & (s == NS - 1))
    def _():
        pltpu.sync_copy(z2d_hbm.at[pl.ds(0, T_LAST)],
                        acc_sh.at[pl.ds(off, T_LAST)])

    @pl.when((c == 1) & (s != NS - 1))
    def _():
        pltpu.sync_copy(z2d_hbm.at[pl.ds(0, T_CHUNK)],
                        acc_sh.at[pl.ds(off, T_CHUNK)])

    plsc.subcore_barrier()

    erow0 = g * RW
    gsems = (gsem0, gsem1)
    ssems = (ssem0, ssem1)

    def drain(sem, k):
        pltpu.make_async_copy(xs_hbm.at[pl.ds(0, 128)],
                              rows_v.at[k], sem).wait()

    for half in range(2):
        hbase = erow0 + half * HLF
        pltpu.sync_copy(s2d.at[pl.ds(hbase, HLF)], sidx_v)
        pltpu.sync_copy(r2d.at[pl.ds(hbase, HLF)], ridx_v)
        pltpu.async_copy(xs_hbm.at[sidx_v.at[0]], rows_v.at[0], gsem0)

        def ring(i, _):
            for j in range(8):
                r = i * 8 + j
                k = j % 2
                drain(gsems[k], k)
                pltpu.async_copy(rows_v.at[k], acc_sh.at[ridx_v.at[r]],
                                 ssems[k], add=True)

                @pl.when(r + 1 < HLF)
                def _():
                    @pl.when(r >= 1)
                    def _():
                        drain(ssems[1 - k], 1 - k)

                    pltpu.async_copy(xs_hbm.at[sidx_v.at[r + 1]],
                                     rows_v.at[1 - k], gsems[1 - k])
            return 0

        lax.fori_loop(0, HLF // 8, ring, 0)
        drain(ssems[0], 0)
        drain(ssems[1], 1)

    plsc.subcore_barrier()

    @pl.when((c == 0) & (s == NS - 1))
    def _():
        pltpu.sync_copy(acc_sh.at[pl.ds(off, T_LAST)],
                        out_a.at[pl.ds(off, T_LAST)])

    @pl.when((c == 0) & (s != NS - 1))
    def _():
        pltpu.sync_copy(acc_sh.at[pl.ds(off, T_CHUNK)],
                        out_a.at[pl.ds(off, T_CHUNK)])

    @pl.when((c == 1) & (s == NS - 1))
    def _():
        pltpu.sync_copy(acc_sh.at[pl.ds(off, T_LAST)],
                        out_b.at[pl.ds(off, T_LAST)])

    @pl.when((c == 1) & (s != NS - 1))
    def _():
        pltpu.sync_copy(acc_sh.at[pl.ds(off, T_CHUNK)],
                        out_b.at[pl.ds(off, T_CHUNK)])


def _add_body(a_ref, b_ref, out_ref):
    out_ref[...] = a_ref[...] + b_ref[...]


def _add_tc(a, b):
    BR = 1000
    return pl.pallas_call(
        _add_body,
        grid=(N // BR,),
        in_specs=[pl.BlockSpec((BR, D), lambda i: (i, 0)),
                  pl.BlockSpec((BR, D), lambda i: (i, 0))],
        out_specs=pl.BlockSpec((BR, D), lambda i: (i, 0)),
        out_shape=jax.ShapeDtypeStruct((N, D), jnp.float32),
    )(a, b)


def kernel(nodes, senders, receivers, W):
    ar = jnp.arange(EPAD, dtype=jnp.int32)
    pad_trash = N + (ar % 64)

    def interleave(x, pad):
        xw = x.reshape(NW, E // NW)
        pw = pad.reshape(NW, EPAD // NW)
        return jnp.concatenate([xw, pw], axis=1).reshape(EROWS, 128)

    s2dp = interleave(senders, pad_trash)
    r2dp = interleave(receivers, pad_trash)

    nodes_pad = jnp.concatenate([nodes, jnp.zeros((HB - N, D), jnp.float32)])
    ones128 = jnp.ones((128,), jnp.float32)
    z1d = jnp.zeros((HT,), jnp.float32)
    z2d = jnp.zeros((T_LAST, D), jnp.float32)
    eye = jnp.eye(128, dtype=jnp.float32)

    x = _matmul_tc(nodes_pad, W)
    hs_a, hr_a, hs_b, hr_b = _hist_sc(s2dp, r2dp, ones128, z1d)

    xs = _scale_tc(x,
                   hs_a.reshape(80, 128), hs_b.reshape(80, 128),
                   hr_a.reshape(80, 128), hr_b.reshape(80, 128), eye)
    pa, pb = _message_sc(xs, s2dp, r2dp, z2d)
    return _add_tc(pa, pb)


# speedup vs baseline: 16.5042x; 1.0512x over previous
import functools

import jax
import jax.numpy as jnp
from jax import lax
from jax.experimental import pallas as pl
from jax.experimental.pallas import tpu as pltpu
from jax.experimental.pallas import tpu_sc as plsc

N = 10000
E = 320000
D = 128

NC = 2
NS = 16
NW = NC * NS

EROWS = 2560
EPAD = EROWS * 128 - E
RW = EROWS // NW
CHUNK = 8
RB = 2
HLF = RW // 2

NB = N + 64
HB = 10240
HT = HB // NS

T_CHUNK = 624
T_LAST = N - 15 * T_CHUNK

_mesh = plsc.VectorSubcoreMesh(core_axis_name="c", subcore_axis_name="s",
                               num_cores=NC, num_subcores=NS)


@functools.partial(
    pl.kernel,
    out_type=(jax.ShapeDtypeStruct((HB,), jnp.float32),
              jax.ShapeDtypeStruct((HB,), jnp.float32),
              jax.ShapeDtypeStruct((HB,), jnp.float32),
              jax.ShapeDtypeStruct((HB,), jnp.float32)),
    mesh=_mesh,
    scratch_types=[
        pltpu.VMEM((RW, 128), jnp.int32),
        pltpu.VMEM((RW, 128), jnp.int32),
        pltpu.VMEM((128,), jnp.float32),
        pltpu.VMEM_SHARED((HB,), jnp.float32),
        pltpu.VMEM_SHARED((HB,), jnp.float32),
        pltpu.SemaphoreType.DMA,
    ],
)
def _hist_sc(s2d, r2d, ones_hbm, z1d_hbm, hs_a, hr_a, hs_b, hr_b,
             sidx_v, ridx_v, ones_v, hs_sh, hr_sh, sem):
    c = lax.axis_index("c")
    s = lax.axis_index("s")
    g = s * NC + c

    off = s * HT
    pltpu.sync_copy(z1d_hbm, hs_sh.at[pl.ds(off, HT)])
    pltpu.sync_copy(z1d_hbm, hr_sh.at[pl.ds(off, HT)])
    pltpu.sync_copy(ones_hbm, ones_v)
    plsc.subcore_barrier()

    row0 = g * RW
    pltpu.sync_copy(s2d.at[pl.ds(row0, RW)], sidx_v)
    pltpu.sync_copy(r2d.at[pl.ds(row0, RW)], ridx_v)

    def fire(i, _):
        for j in range(CHUNK):
            r = i * CHUNK + j
            pltpu.async_copy(ones_v, hs_sh.at[sidx_v.at[r]], sem, add=True)
            pltpu.async_copy(ones_v, hr_sh.at[ridx_v.at[r]], sem, add=True)
        return 0

    lax.fori_loop(0, RW // CHUNK, fire, 0)

    def drn(i, _):
        for j in range(2 * CHUNK):
            pltpu.make_async_copy(z1d_hbm.at[pl.ds(0, 128)], ones_v,
                                  sem).wait()
        return 0

    lax.fori_loop(0, RW // CHUNK, drn, 0)
    plsc.subcore_barrier()

    @pl.when(c == 0)
    def _():
        pltpu.sync_copy(hs_sh.at[pl.ds(off, HT)], hs_a.at[pl.ds(off, HT)])
        pltpu.sync_copy(hr_sh.at[pl.ds(off, HT)], hr_a.at[pl.ds(off, HT)])

    @pl.when(c == 1)
    def _():
        pltpu.sync_copy(hs_sh.at[pl.ds(off, HT)], hs_b.at[pl.ds(off, HT)])
        pltpu.sync_copy(hr_sh.at[pl.ds(off, HT)], hr_b.at[pl.ds(off, HT)])


def _matmul_body(nodes_ref, w_ref, out_ref):
    out_ref[...] = jnp.dot(nodes_ref[...], w_ref[...],
                           preferred_element_type=jnp.float32)


def _matmul_tc(nodes_pad, W):
    BR = 1024
    return pl.pallas_call(
        _matmul_body,
        grid=(HB // BR,),
        in_specs=[pl.BlockSpec((BR, D), lambda i: (i, 0)),
                  pl.BlockSpec((D, D), lambda i: (0, 0))],
        out_specs=pl.BlockSpec((BR, D), lambda i: (i, 0)),
        out_shape=jax.ShapeDtypeStruct((HB, D), jnp.float32),
    )(nodes_pad, W)


def _scale_body(x_ref, hs0_ref, hs1_ref, hr0_ref, hr1_ref, eye_ref, out_ref):
    cs = hs0_ref[...] + hs1_ref[...]
    cr = hr0_ref[...] + hr1_ref[...]
    scale = lax.rsqrt((2.0 * cs + 2.0) * (2.0 * cr + 2.0))
    for j in range(8):
        diag = eye_ref[...] * scale[j:j + 1, :]
        out_ref[pl.ds(j * 128, 128), :] = jnp.dot(
            diag, x_ref[pl.ds(j * 128, 128), :],
            preferred_element_type=jnp.float32,
            precision=lax.Precision.HIGHEST)


def _scale_tc(x, hs0, hs1, hr0, hr1, eye):
    BR = 1024
    return pl.pallas_call(
        _scale_body,
        grid=(HB // BR,),
        in_specs=[
            pl.BlockSpec((BR, D), lambda i: (i, 0)),
            pl.BlockSpec((8, 128), lambda i: (i, 0)),
            pl.BlockSpec((8, 128), lambda i: (i, 0)),
            pl.BlockSpec((8, 128), lambda i: (i, 0)),
            pl.BlockSpec((8, 128), lambda i: (i, 0)),
            pl.BlockSpec((128, 128), lambda i: (0, 0)),
        ],
        out_specs=pl.BlockSpec((BR, D), lambda i: (i, 0)),
        out_shape=jax.ShapeDtypeStruct((HB, D), jnp.float32),
    )(x, hs0, hs1, hr0, hr1, eye)


@functools.partial(
    pl.kernel,
    out_type=(jax.ShapeDtypeStruct((N, D), jnp.float32),
              jax.ShapeDtypeStruct((N, D), jnp.float32)),
    mesh=_mesh,
    scratch_types=[
        pltpu.VMEM((HLF, 128), jnp.int32),
        pltpu.VMEM((HLF, 128), jnp.int32),
        pltpu.VMEM((RB, 128, D), jnp.float32),
        pltpu.VMEM_SHARED((NB, D), jnp.float32),
        pltpu.SemaphoreType.DMA,
        pltpu.SemaphoreType.DMA,
        pltpu.SemaphoreType.DMA,
        pltpu.SemaphoreType.DMA,
    ],
)
def _message_sc(xs_hbm, s2d, r2d, z2d_hbm, out_a, out_b,
                sidx_v, ridx_v, rows_v, acc_sh, gsem0, gsem1, ssem0, ssem1):
    c = lax.axis_index("c")
    s = lax.axis_index("s")
    g = s * NC + c

    off = s * T_CHUNK

    @pl.when((c == 0) & (s == NS - 1))
    def _():
        pltpu.sync_copy(xs_hbm.at[pl.ds(off, T_LAST)],
                        acc_sh.at[pl.ds(off, T_LAST)])

    @pl.when((c == 0) & (s != NS - 1))
    def _():
        pltpu.sync_copy(xs_hbm.at[pl.ds(off, T_CHUNK)],
                        acc_sh.at[pl.ds(off, T_CHUNK)])

    @pl.when((c == 1) & (s == NS - 1))
    def _():
        pltpu.sync_copy(z2d_hbm.at[pl.ds(0, T_LAST)],
                        acc_sh.at[pl.ds(off, T_LAST)])

    @pl.when((c == 1) & (s != NS - 1))
    def _():
        pltpu.sync_copy(z2d_hbm.at[pl.ds(0, T_CHUNK)],
                        acc_sh.at[pl.ds(off, T_CHUNK)])

    plsc.subcore_barrier()

    erow0 = g * RW
    gsems = (gsem0, gsem1)
    ssems = (ssem0, ssem1)

    def drain(sem, k):
        pltpu.make_async_copy(xs_hbm.at[pl.ds(0, 128)],
                              rows_v.at[k], sem).wait()

    for half in range(2):
        hbase = erow0 + half * HLF
        pltpu.sync_copy(s2d.at[pl.ds(hbase, HLF)], sidx_v)
        pltpu.sync_copy(r2d.at[pl.ds(hbase, HLF)], ridx_v)
        pltpu.async_copy(xs_hbm.at[sidx_v.at[0]], rows_v.at[0], gsem0)

        def ring(i, _):
            for j in range(8):
                r = i * 8 + j
                k = j % 2
                drain(gsems[k], k)
                pltpu.async_copy(rows_v.at[k], acc_sh.at[ridx_v.at[r]],
                                 ssems[k], add=True)

                @pl.when(r + 1 < HLF)
                def _():
                    @pl.when(r >= 1)
                    def _():
                        drain(ssems[1 - k], 1 - k)

                    pltpu.async_copy(xs_hbm.at[sidx_v.at[r + 1]],
                                     rows_v.at[1 - k], gsems[1 - k])
            return 0

        lax.fori_loop(0, HLF // 8, ring, 0)
        drain(ssems[0], 0)
        drain(ssems[1], 1)

    plsc.subcore_barrier()

    @pl.when((c == 0) & (s == NS - 1))
    def _():
        pltpu.sync_copy(acc_sh.at[pl.ds(off, T_LAST)],
                        out_a.at[pl.ds(off, T_LAST)])

    @pl.when((c == 0) & (s != NS - 1))
    def _():
        pltpu.sync_copy(acc_sh.at[pl.ds(off, T_CHUNK)],
                        out_a.at[pl.ds(off, T_CHUNK)])

    @pl.when((c == 1) & (s == NS - 1))
    def _():
        pltpu.sync_copy(acc_sh.at[pl.ds(off, T_LAST)],
                        out_b.at[pl.ds(off, T_LAST)])

    @pl.when((c == 1) & (s != NS - 1))
    def _():
        pltpu.sync_copy(acc_sh.at[pl.ds(off, T_CHUNK)],
                        out_b.at[pl.ds(off, T_CHUNK)])


def _add_body(a_ref, b_ref, out_ref):
    out_ref[...] = a_ref[...] + b_ref[...]


def _add_tc(a, b):
    BR = 1000
    return pl.pallas_call(
        _add_body,
        grid=(N // BR,),
        in_specs=[pl.BlockSpec((BR, D), lambda i: (i, 0)),
                  pl.BlockSpec((BR, D), lambda i: (i, 0))],
        out_specs=pl.BlockSpec((BR, D), lambda i: (i, 0)),
        out_shape=jax.ShapeDtypeStruct((N, D), jnp.float32),
    )(a, b)


def kernel(nodes, senders, receivers, W):
    ar = jnp.arange(EPAD, dtype=jnp.int32)
    pad_trash = N + (ar % 64)

    def interleave(x, pad):
        xw = x.reshape(NW, E // NW)
        pw = pad.reshape(NW, EPAD // NW)
        return jnp.concatenate([xw, pw], axis=1).reshape(EROWS, 128)

    s2dp = interleave(senders, pad_trash)
    r2dp = interleave(receivers, pad_trash)

    nodes_pad = jnp.concatenate([nodes, jnp.zeros((HB - N, D), jnp.float32)])
    ones128 = jnp.ones((128,), jnp.float32)
    z1d = jnp.zeros((HT,), jnp.float32)
    z2d = jnp.zeros((T_LAST, D), jnp.float32)
    eye = jnp.eye(128, dtype=jnp.float32)

    x = _matmul_tc(nodes_pad, W)
    hs_a, hr_a, hs_b, hr_b = _hist_sc(s2dp, r2dp, ones128, z1d)

    xs = _scale_tc(x,
                   hs_a.reshape(80, 128), hs_b.reshape(80, 128),
                   hr_a.reshape(80, 128), hr_b.reshape(80, 128), eye)
    pa, pb = _message_sc(xs, s2dp, r2dp, z2d)
    return _add_tc(pa, pb)


# speedup vs baseline: 16.6334x; 1.0078x over previous
import functools

import jax
import jax.numpy as jnp
from jax import lax
from jax.experimental import pallas as pl
from jax.experimental.pallas import tpu as pltpu
from jax.experimental.pallas import tpu_sc as plsc

N = 10000
E = 320000
D = 128

NC = 2
NS = 16
NW = NC * NS

EROWS = 2560
EPAD = EROWS * 128 - E
RW = EROWS // NW
CHUNK = 8
RB = 2
HLF = RW // 2

NB = N + 64
HB = 10240
HT = HB // NS

T_CHUNK = 624
T_LAST = N - 15 * T_CHUNK

_mesh = plsc.VectorSubcoreMesh(core_axis_name="c", subcore_axis_name="s",
                               num_cores=NC, num_subcores=NS)


@functools.partial(
    pl.kernel,
    out_type=(jax.ShapeDtypeStruct((HB,), jnp.float32),
              jax.ShapeDtypeStruct((HB,), jnp.float32),
              jax.ShapeDtypeStruct((HB,), jnp.float32),
              jax.ShapeDtypeStruct((HB,), jnp.float32)),
    mesh=_mesh,
    scratch_types=[
        pltpu.VMEM((RW, 128), jnp.int32),
        pltpu.VMEM((RW, 128), jnp.int32),
        pltpu.VMEM((128,), jnp.float32),
        pltpu.VMEM_SHARED((HB,), jnp.float32),
        pltpu.VMEM_SHARED((HB,), jnp.float32),
        pltpu.SemaphoreType.DMA,
    ],
)
def _hist_sc(s2d, r2d, ones_hbm, z1d_hbm, hs_a, hr_a, hs_b, hr_b,
             sidx_v, ridx_v, ones_v, hs_sh, hr_sh, sem):
    c = lax.axis_index("c")
    s = lax.axis_index("s")
    g = s * NC + c

    off = s * HT
    pltpu.sync_copy(z1d_hbm, hs_sh.at[pl.ds(off, HT)])
    pltpu.sync_copy(z1d_hbm, hr_sh.at[pl.ds(off, HT)])
    pltpu.sync_copy(ones_hbm, ones_v)
    plsc.subcore_barrier()

    row0 = g * RW
    pltpu.sync_copy(s2d.at[pl.ds(row0, RW)], sidx_v)
    pltpu.sync_copy(r2d.at[pl.ds(row0, RW)], ridx_v)

    def fire(i, _):
        for j in range(CHUNK):
            r = i * CHUNK + j
            pltpu.async_copy(ones_v, hs_sh.at[sidx_v.at[r]], sem, add=True)
            pltpu.async_copy(ones_v, hr_sh.at[ridx_v.at[r]], sem, add=True)
        return 0

    lax.fori_loop(0, RW // CHUNK, fire, 0)

    def drn(i, _):
        for j in range(2 * CHUNK):
            pltpu.make_async_copy(z1d_hbm.at[pl.ds(0, 128)], ones_v,
                                  sem).wait()
        return 0

    lax.fori_loop(0, RW // CHUNK, drn, 0)
    plsc.subcore_barrier()

    @pl.when(c == 0)
    def _():
        pltpu.sync_copy(hs_sh.at[pl.ds(off, HT)], hs_a.at[pl.ds(off, HT)])
        pltpu.sync_copy(hr_sh.at[pl.ds(off, HT)], hr_a.at[pl.ds(off, HT)])

    @pl.when(c == 1)
    def _():
        pltpu.sync_copy(hs_sh.at[pl.ds(off, HT)], hs_b.at[pl.ds(off, HT)])
        pltpu.sync_copy(hr_sh.at[pl.ds(off, HT)], hr_b.at[pl.ds(off, HT)])


def _matmul_body(nodes_ref, w_ref, out_ref):
    out_ref[...] = jnp.dot(nodes_ref[...], w_ref[...],
                           preferred_element_type=jnp.float32)


def _matmul_tc(nodes_pad, W):
    BR = 1024
    return pl.pallas_call(
        _matmul_body,
        grid=(HB // BR,),
        in_specs=[pl.BlockSpec((BR, D), lambda i: (i, 0)),
                  pl.BlockSpec((D, D), lambda i: (0, 0))],
        out_specs=pl.BlockSpec((BR, D), lambda i: (i, 0)),
        out_shape=jax.ShapeDtypeStruct((HB, D), jnp.float32),
    )(nodes_pad, W)


def _scale_body(x_ref, hs0_ref, hs1_ref, hr0_ref, hr1_ref, eye_ref, out_ref):
    cs = hs0_ref[...] + hs1_ref[...]
    cr = hr0_ref[...] + hr1_ref[...]
    scale = lax.rsqrt((2.0 * cs + 2.0) * (2.0 * cr + 2.0))
    for j in range(8):
        diag = eye_ref[...] * scale[j:j + 1, :]
        out_ref[pl.ds(j * 128, 128), :] = jnp.dot(
            diag, x_ref[pl.ds(j * 128, 128), :],
            preferred_element_type=jnp.float32,
            precision=lax.Precision.HIGHEST)


def _scale_tc(x, hs0, hs1, hr0, hr1, eye):
    BR = 1024
    return pl.pallas_call(
        _scale_body,
        grid=(HB // BR,),
        in_specs=[
            pl.BlockSpec((BR, D), lambda i: (i, 0)),
            pl.BlockSpec((8, 128), lambda i: (i, 0)),
            pl.BlockSpec((8, 128), lambda i: (i, 0)),
            pl.BlockSpec((8, 128), lambda i: (i, 0)),
            pl.BlockSpec((8, 128), lambda i: (i, 0)),
            pl.BlockSpec((128, 128), lambda i: (0, 0)),
        ],
        out_specs=pl.BlockSpec((BR, D), lambda i: (i, 0)),
        out_shape=jax.ShapeDtypeStruct((HB, D), jnp.float32),
    )(x, hs0, hs1, hr0, hr1, eye)


@functools.partial(
    pl.kernel,
    out_type=(jax.ShapeDtypeStruct((N, D), jnp.float32),
              jax.ShapeDtypeStruct((N, D), jnp.float32)),
    mesh=_mesh,
    scratch_types=[
        pltpu.VMEM((HLF, 128), jnp.int32),
        pltpu.VMEM((HLF, 128), jnp.int32),
        pltpu.VMEM((RB, 128, D), jnp.float32),
        pltpu.VMEM_SHARED((NB, D), jnp.float32),
        pltpu.SemaphoreType.DMA,
        pltpu.SemaphoreType.DMA,
        pltpu.SemaphoreType.DMA,
        pltpu.SemaphoreType.DMA,
    ],
)
def _message_sc(xs_hbm, s2d, r2d, z2d_hbm, out_a, out_b,
                sidx_v, ridx_v, rows_v, acc_sh, gsem0, gsem1, ssem0, ssem1):
    c = lax.axis_index("c")
    s = lax.axis_index("s")
    g = s * NC + c

    off = s * T_CHUNK
    erow0 = g * RW

    pltpu.sync_copy(s2d.at[pl.ds(erow0, HLF)], sidx_v)
    pltpu.sync_copy(r2d.at[pl.ds(erow0, HLF)], ridx_v)
    pltpu.async_copy(xs_hbm.at[sidx_v.at[0]], rows_v.at[0], gsem0)

    @pl.when((c == 0) & (s == NS - 1))
    def _():
        pltpu.sync_copy(xs_hbm.at[pl.ds(off, T_LAST)],
                        acc_sh.at[pl.ds(off, T_LAST)])

    @pl.when((c == 0) & (s != NS - 1))
    def _():
        pltpu.sync_copy(xs_hbm.at[pl.ds(off, T_CHUNK)],
                        acc_sh.at[pl.ds(off, T_CHUNK)])

    @pl.when((c == 1) & (s == NS - 1))
    def _():
        pltpu.sync_copy(z2d_hbm.at[pl.ds(0, T_LAST)],
                        acc_sh.at[pl.ds(off, T_LAST)])

    @pl.when((c == 1) & (s != NS - 1))
    def _():
        pltpu.sync_copy(z2d_hbm.at[pl.ds(0, T_CHUNK)],
                        acc_sh.at[pl.ds(off, T_CHUNK)])

    plsc.subcore_barrier()

    gsems = (gsem0, gsem1)
    ssems = (ssem0, ssem1)

    def drain(sem, k):
        pltpu.make_async_copy(xs_hbm.at[pl.ds(0, 128)],
                              rows_v.at[k], sem).wait()

    for half in range(2):
        if half == 1:
            pltpu.sync_copy(s2d.at[pl.ds(erow0 + HLF, HLF)], sidx_v)
            pltpu.sync_copy(r2d.at[pl.ds(erow0 + HLF, HLF)], ridx_v)
            pltpu.async_copy(xs_hbm.at[sidx_v.at[0]], rows_v.at[0], gsem0)

        def ring(i, _):
            for j in range(8):
                r = i * 8 + j
                k = j % 2
                drain(gsems[k], k)
                pltpu.async_copy(rows_v.at[k], acc_sh.at[ridx_v.at[r]],
                                 ssems[k], add=True)

                @pl.when(r + 1 < HLF)
                def _():
                    @pl.when(r >= 1)
                    def _():
                        drain(ssems[1 - k], 1 - k)

                    pltpu.async_copy(xs_hbm.at[sidx_v.at[r + 1]],
                                     rows_v.at[1 - k], gsems[1 - k])
            return 0

        lax.fori_loop(0, HLF // 8, ring, 0)
        drain(ssems[0], 0)
        drain(ssems[1], 1)

    plsc.subcore_barrier()

    @pl.when((c == 0) & (s == NS - 1))
    def _():
        pltpu.sync_copy(acc_sh.at[pl.ds(off, T_LAST)],
                        out_a.at[pl.ds(off, T_LAST)])

    @pl.when((c == 0) & (s != NS - 1))
    def _():
        pltpu.sync_copy(acc_sh.at[pl.ds(off, T_CHUNK)],
                        out_a.at[pl.ds(off, T_CHUNK)])

    @pl.when((c == 1) & (s == NS - 1))
    def _():
        pltpu.sync_copy(acc_sh.at[pl.ds(off, T_LAST)],
                        out_b.at[pl.ds(off, T_LAST)])

    @pl.when((c == 1) & (s != NS - 1))
    def _():
        pltpu.sync_copy(acc_sh.at[pl.ds(off, T_CHUNK)],
                        out_b.at[pl.ds(off, T_CHUNK)])


def _add_body(a_ref, b_ref, out_ref):
    out_ref[...] = a_ref[...] + b_ref[...]


def _add_tc(a, b):
    BR = 1000
    return pl.pallas_call(
        _add_body,
        grid=(N // BR,),
        in_specs=[pl.BlockSpec((BR, D), lambda i: (i, 0)),
                  pl.BlockSpec((BR, D), lambda i: (i, 0))],
        out_specs=pl.BlockSpec((BR, D), lambda i: (i, 0)),
        out_shape=jax.ShapeDtypeStruct((N, D), jnp.float32),
    )(a, b)


def kernel(nodes, senders, receivers, W):
    ar = jnp.arange(EPAD, dtype=jnp.int32)
    pad_trash = N + (ar % 64)

    def interleave(x, pad):
        xw = x.reshape(NW, E // NW)
        pw = pad.reshape(NW, EPAD // NW)
        return jnp.concatenate([xw, pw], axis=1).reshape(EROWS, 128)

    s2dp = interleave(senders, pad_trash)
    r2dp = interleave(receivers, pad_trash)

    nodes_pad = jnp.concatenate([nodes, jnp.zeros((HB - N, D), jnp.float32)])
    ones128 = jnp.ones((128,), jnp.float32)
    z1d = jnp.zeros((HT,), jnp.float32)
    z2d = jnp.zeros((T_LAST, D), jnp.float32)
    eye = jnp.eye(128, dtype=jnp.float32)

    x = _matmul_tc(nodes_pad, W)
    hs_a, hr_a, hs_b, hr_b = _hist_sc(s2dp, r2dp, ones128, z1d)

    xs = _scale_tc(x,
                   hs_a.reshape(80, 128), hs_b.reshape(80, 128),
                   hr_a.reshape(80, 128), hr_b.reshape(80, 128), eye)
    pa, pb = _message_sc(xs, s2dp, r2dp, z2d)
    return _add_tc(pa, pb)
